# baseline (speedup 1.0000x reference)
import functools

import jax
import jax.numpy as jnp
from jax import lax
from jax.experimental import pallas as pl
from jax.experimental.pallas import tpu as pltpu
from jax.experimental.pallas import tpu_sc as plsc

N = 10000
E = 320000
C = 128
K = 9

NC = 2
NS = 16
NW = NC * NS
CH2 = C // NC
EPT = E // NS
G = 125
CH = EPT // G
NP = 10240
RPT = NP // NS

BN = 1000
NB = N // BN


def _mm_body(x_ref, w_ref, y_ref):
    d = jnp.dot(x_ref[...], w_ref[0], preferred_element_type=jnp.float32)
    y_ref[0, 0] = d[:, :CH2]
    y_ref[1, 0] = d[:, CH2:]


def _transform(h, W):
    return pl.pallas_call(
        _mm_body,
        grid=(K, NB),
        in_specs=[
            pl.BlockSpec((BN, C), lambda k, n: (n, 0)),
            pl.BlockSpec((1, C, C), lambda k, n: (k, 0, 0)),
        ],
        out_specs=pl.BlockSpec((NC, 1, BN, CH2), lambda k, n: (0, k, n, 0)),
        out_shape=jax.ShapeDtypeStruct((NC, K, N, CH2), jnp.float32),
    )(h, W)


def _ln(h, g, b):
    mu = jnp.mean(h, axis=-1, keepdims=True)
    var = jnp.mean((h - mu) ** 2, axis=-1, keepdims=True)
    return (h - mu) * lax.rsqrt(var + 1e-6) * g + b


def _mid_body(p_ref, g_ref, b_ref, w_ref, y_ref):
    h = jnp.concatenate([p_ref[0], p_ref[1]], axis=-1)
    h = jax.nn.gelu(_ln(h, g_ref[...], b_ref[...]))
    for k in range(K):
        d = jnp.dot(h, w_ref[k], preferred_element_type=jnp.float32)
        y_ref[0, k] = d[:, :CH2]
        y_ref[1, k] = d[:, CH2:]


def _mid(parts, g, b, W):
    return pl.pallas_call(
        _mid_body,
        grid=(NB,),
        in_specs=[
            pl.BlockSpec((NC, BN, CH2), lambda n: (0, n, 0)),
            pl.BlockSpec((1, C), lambda n: (0, 0)),
            pl.BlockSpec((1, C), lambda n: (0, 0)),
            pl.BlockSpec((K, C, C), lambda n: (0, 0, 0)),
        ],
        out_specs=pl.BlockSpec((NC, K, BN, CH2), lambda n: (0, 0, n, 0)),
        out_shape=jax.ShapeDtypeStruct((NC, K, N, CH2), jnp.float32),
    )(parts, g, b, W)


def _final_body(p_ref, g_ref, b_ref, x_ref, o_ref):
    h = jnp.concatenate([p_ref[0], p_ref[1]], axis=-1)
    h = _ln(h, g_ref[...], b_ref[...]) + x_ref[...]
    o_ref[...] = jax.nn.gelu(h)


def _final(parts, g, b, x):
    return pl.pallas_call(
        _final_body,
        grid=(NB,),
        in_specs=[
            pl.BlockSpec((NC, BN, CH2), lambda n: (0, n, 0)),
            pl.BlockSpec((1, C), lambda n: (0, 0)),
            pl.BlockSpec((1, C), lambda n: (0, 0)),
            pl.BlockSpec((BN, C), lambda n: (n, 0)),
        ],
        out_specs=pl.BlockSpec((BN, C), lambda n: (n, 0)),
        out_shape=jax.ShapeDtypeStruct((N, C), jnp.float32),
    )(parts, g, b, x)


@functools.cache
def _make_sc_conv():
    mesh = plsc.VectorSubcoreMesh(core_axis_name="c", subcore_axis_name="s")

    @functools.partial(
        pl.kernel,
        out_type=jax.ShapeDtypeStruct((NC, NP, CH2), jnp.float32),
        mesh=mesh,
        compiler_params=pltpu.CompilerParams(use_tc_tiling_on_sc=False),
        scratch_types=[
            pltpu.VMEM((2, CH, G), jnp.int32),
            pltpu.VMEM((G, CH2), jnp.float32),
            pltpu.VMEM((G, CH2), jnp.float32),
            pltpu.VMEM_SHARED((NP, CH2), jnp.float32),
            pltpu.SemaphoreType.DMA,
            pltpu.SemaphoreType.DMA,
        ],
    )
    def sc_conv(y_hbm, eidx_hbm, zeros_hbm, out_hbm,
                idx_v, slot0, slot1, acc_s, sem0, sem1):
        c = lax.axis_index("c")
        s = lax.axis_index("s")
        wid = c * NS + s
        pltpu.sync_copy(zeros_hbm.at[pl.ds(s * RPT, RPT)],
                        acc_s.at[pl.ds(s * RPT, RPT)])
        pltpu.sync_copy(eidx_hbm.at[wid], idx_v)
        plsc.subcore_barrier()

        gix = idx_v.at[0]
        dix = idx_v.at[1]
        pltpu.async_copy(y_hbm.at[gix.at[0]], slot0, sem0)
        pltpu.async_copy(y_hbm.at[gix.at[1]], slot1, sem1)

        def chunk(jj, carry):
            j = 2 * jj
            jp2 = lax.rem(j + 2, CH)
            jp3 = lax.rem(j + 3, CH)
            pltpu.make_async_copy(y_hbm.at[gix.at[0]], slot0, sem0).wait()
            pltpu.sync_copy(slot0, acc_s.at[dix.at[j]], add=True)
            pltpu.async_copy(y_hbm.at[gix.at[jp2]], slot0, sem0)
            pltpu.make_async_copy(y_hbm.at[gix.at[1]], slot1, sem1).wait()
            pltpu.sync_copy(slot1, acc_s.at[dix.at[j + 1]], add=True)
            pltpu.async_copy(y_hbm.at[gix.at[jp3]], slot1, sem1)
            return carry

        lax.fori_loop(0, CH // 2, chunk, 0)
        pltpu.make_async_copy(y_hbm.at[gix.at[0]], slot0, sem0).wait()
        pltpu.make_async_copy(y_hbm.at[gix.at[1]], slot1, sem1).wait()
        plsc.subcore_barrier()
        pltpu.sync_copy(acc_s.at[pl.ds(s * RPT, RPT)],
                        out_hbm.at[c, pl.ds(s * RPT, RPT)])

    return sc_conv


def _sc_conv(y, eidx, zeros):
    return _make_sc_conv()(y, eidx, zeros)


def kernel(x, edge_index, kernel_ids, W1, g1, b1, W2, g2, b2):
    src = edge_index[0].astype(jnp.int32)
    dst = edge_index[1].astype(jnp.int32)
    gidx = (kernel_ids.astype(jnp.int32) * N + src).reshape(1, NS, 1, CH, G)
    gidx = jnp.concatenate([gidx, gidx + K * N], axis=0)
    didx = jnp.broadcast_to(dst.reshape(1, NS, 1, CH, G), (NC, NS, 1, CH, G))
    eidx = jnp.concatenate([gidx, didx], axis=2).reshape(NW, 2, CH, G)
    zeros = jnp.zeros((NP, CH2), jnp.float32)
    g1r, b1r = g1.reshape(1, C), b1.reshape(1, C)
    g2r, b2r = g2.reshape(1, C), b2.reshape(1, C)

    y1 = _transform(x, W1).reshape(NC * K * N, CH2)
    p1 = _sc_conv(y1, eidx, zeros)
    y2 = _mid(p1, g1r, b1r, W2).reshape(NC * K * N, CH2)
    p2 = _sc_conv(y2, eidx, zeros)
    return _final(p2, g2r, b2r, x)


# speedup vs baseline: 1.3694x; 1.3694x over previous
import functools

import jax
import jax.numpy as jnp
from jax import lax
from jax.experimental import pallas as pl
from jax.experimental.pallas import tpu as pltpu
from jax.experimental.pallas import tpu_sc as plsc

N = 10000
E = 320000
C = 128
K = 9

NC = 2
NS = 16
NW = NC * NS
CH2 = C // NC
EPT = E // NS
G = 125
CH = EPT // G
NP = 10240
RPT = NP // NS

BN = 1000
NB = N // BN


def _mm_body(x_ref, w_ref, y_ref):
    y_ref[0] = jnp.dot(x_ref[...], w_ref[0],
                       preferred_element_type=jnp.float32)


def _transform(h, W):
    return pl.pallas_call(
        _mm_body,
        grid=(K, NB),
        in_specs=[
            pl.BlockSpec((BN, C), lambda k, n: (n, 0)),
            pl.BlockSpec((1, C, C), lambda k, n: (k, 0, 0)),
        ],
        out_specs=pl.BlockSpec((1, BN, C), lambda k, n: (k, n, 0)),
        out_shape=jax.ShapeDtypeStruct((K, N, C), jnp.float32),
    )(h, W)


def _ln(h, g, b):
    mu = jnp.mean(h, axis=-1, keepdims=True)
    var = jnp.mean((h - mu) ** 2, axis=-1, keepdims=True)
    return (h - mu) * lax.rsqrt(var + 1e-6) * g + b


def _mid_body(p_ref, g_ref, b_ref, w_ref, y_ref):
    h = jnp.concatenate([p_ref[0], p_ref[1]], axis=-1)
    h = jax.nn.gelu(_ln(h, g_ref[...], b_ref[...]))
    for k in range(K):
        y_ref[k] = jnp.dot(h, w_ref[k], preferred_element_type=jnp.float32)


def _mid(parts, g, b, W):
    return pl.pallas_call(
        _mid_body,
        grid=(NB,),
        in_specs=[
            pl.BlockSpec((NC, BN, CH2), lambda n: (0, n, 0)),
            pl.BlockSpec((1, C), lambda n: (0, 0)),
            pl.BlockSpec((1, C), lambda n: (0, 0)),
            pl.BlockSpec((K, C, C), lambda n: (0, 0, 0)),
        ],
        out_specs=pl.BlockSpec((K, BN, C), lambda n: (0, n, 0)),
        out_shape=jax.ShapeDtypeStruct((K, N, C), jnp.float32),
    )(parts, g, b, W)


def _final_body(p_ref, g_ref, b_ref, x_ref, o_ref):
    h = jnp.concatenate([p_ref[0], p_ref[1]], axis=-1)
    h = _ln(h, g_ref[...], b_ref[...]) + x_ref[...]
    o_ref[...] = jax.nn.gelu(h)


def _final(parts, g, b, x):
    return pl.pallas_call(
        _final_body,
        grid=(NB,),
        in_specs=[
            pl.BlockSpec((NC, BN, CH2), lambda n: (0, n, 0)),
            pl.BlockSpec((1, C), lambda n: (0, 0)),
            pl.BlockSpec((1, C), lambda n: (0, 0)),
            pl.BlockSpec((BN, C), lambda n: (n, 0)),
        ],
        out_specs=pl.BlockSpec((BN, C), lambda n: (n, 0)),
        out_shape=jax.ShapeDtypeStruct((N, C), jnp.float32),
    )(parts, g, b, x)


@functools.cache
def _make_sc_conv():
    mesh = plsc.VectorSubcoreMesh(core_axis_name="c", subcore_axis_name="s")

    @functools.partial(
        pl.kernel,
        out_type=jax.ShapeDtypeStruct((NC, NP, CH2), jnp.float32),
        mesh=mesh,
        compiler_params=pltpu.CompilerParams(use_tc_tiling_on_sc=False),
        scratch_types=[
            pltpu.VMEM((2, CH, G), jnp.int32),
            pltpu.VMEM((G, CH2), jnp.float32),
            pltpu.VMEM((G, CH2), jnp.float32),
            pltpu.VMEM_SHARED((NP, CH2), jnp.float32),
            pltpu.SemaphoreType.DMA,
            pltpu.SemaphoreType.DMA,
        ],
    )
    def sc_conv(y_hbm, eidx_hbm, zeros_hbm, out_hbm,
                idx_v, slot0, slot1, acc_s, sem0, sem1):
        c = lax.axis_index("c")
        s = lax.axis_index("s")
        wid = c * NS + s
        pltpu.sync_copy(zeros_hbm.at[pl.ds(s * RPT, RPT)],
                        acc_s.at[pl.ds(s * RPT, RPT)])
        pltpu.sync_copy(eidx_hbm.at[wid], idx_v)
        plsc.subcore_barrier()

        gix = idx_v.at[0]
        dix = idx_v.at[1]
        pltpu.async_copy(y_hbm.at[gix.at[0]], slot0, sem0)
        pltpu.async_copy(y_hbm.at[gix.at[1]], slot1, sem1)

        def chunk(jj, carry):
            j = 2 * jj
            jp2 = lax.rem(j + 2, CH)
            jp3 = lax.rem(j + 3, CH)
            pltpu.make_async_copy(y_hbm.at[gix.at[0]], slot0, sem0).wait()
            pltpu.sync_copy(slot0, acc_s.at[dix.at[j]], add=True)
            pltpu.async_copy(y_hbm.at[gix.at[jp2]], slot0, sem0)
            pltpu.make_async_copy(y_hbm.at[gix.at[1]], slot1, sem1).wait()
            pltpu.sync_copy(slot1, acc_s.at[dix.at[j + 1]], add=True)
            pltpu.async_copy(y_hbm.at[gix.at[jp3]], slot1, sem1)
            return carry

        lax.fori_loop(0, CH // 2, chunk, 0)
        pltpu.make_async_copy(y_hbm.at[gix.at[0]], slot0, sem0).wait()
        pltpu.make_async_copy(y_hbm.at[gix.at[1]], slot1, sem1).wait()
        plsc.subcore_barrier()
        pltpu.sync_copy(acc_s.at[pl.ds(s * RPT, RPT)],
                        out_hbm.at[c, pl.ds(s * RPT, RPT)])

    return sc_conv


def _sc_conv(y, eidx, zeros):
    return _make_sc_conv()(y, eidx, zeros)


def kernel(x, edge_index, kernel_ids, W1, g1, b1, W2, g2, b2):
    src = edge_index[0].astype(jnp.int32)
    dst = edge_index[1].astype(jnp.int32)
    gidx = 2 * (kernel_ids.astype(jnp.int32) * N + src)
    gidx = gidx.reshape(1, NS, 1, CH, G)
    gidx = jnp.concatenate([gidx, gidx + 1], axis=0)
    didx = jnp.broadcast_to(dst.reshape(1, NS, 1, CH, G), (NC, NS, 1, CH, G))
    eidx = jnp.concatenate([gidx, didx], axis=2).reshape(NW, 2, CH, G)
    zeros = jnp.zeros((NP, CH2), jnp.float32)
    g1r, b1r = g1.reshape(1, C), b1.reshape(1, C)
    g2r, b2r = g2.reshape(1, C), b2.reshape(1, C)

    y1 = _transform(x, W1).reshape(NC * K * N, CH2)
    p1 = _sc_conv(y1, eidx, zeros)
    y2 = _mid(p1, g1r, b1r, W2).reshape(NC * K * N, CH2)
    p2 = _sc_conv(y2, eidx, zeros)
    return _final(p2, g2r, b2r, x)


# speedup vs baseline: 1.3847x; 1.0112x over previous
import functools

import jax
import jax.numpy as jnp
from jax import lax
from jax.experimental import pallas as pl
from jax.experimental.pallas import tpu as pltpu
from jax.experimental.pallas import tpu_sc as plsc

N = 10000
E = 320000
C = 128
K = 9

NC = 2
NS = 16
NW = NC * NS
CH2 = C // NC
EPT = E // NS
G = 125
CH = EPT // G
NP = 10240
RPT = NP // NS

BN = 1000
NB = N // BN


def _mm_body(x_ref, w_ref, y_ref):
    y_ref[0] = jnp.dot(x_ref[...].astype(jnp.bfloat16),
                       w_ref[0].astype(jnp.bfloat16),
                       preferred_element_type=jnp.float32)


def _transform(h, W):
    return pl.pallas_call(
        _mm_body,
        grid=(K, NB),
        in_specs=[
            pl.BlockSpec((BN, C), lambda k, n: (n, 0)),
            pl.BlockSpec((1, C, C), lambda k, n: (k, 0, 0)),
        ],
        out_specs=pl.BlockSpec((1, BN, C), lambda k, n: (k, n, 0)),
        out_shape=jax.ShapeDtypeStruct((K, N, C), jnp.float32),
    )(h, W)


def _ln(h, g, b):
    mu = jnp.mean(h, axis=-1, keepdims=True)
    var = jnp.mean((h - mu) ** 2, axis=-1, keepdims=True)
    return (h - mu) * lax.rsqrt(var + 1e-6) * g + b


def _mid_body(p_ref, g_ref, b_ref, w_ref, y_ref):
    h = jnp.concatenate([p_ref[0], p_ref[1]], axis=-1)
    h = jax.nn.gelu(_ln(h, g_ref[...], b_ref[...]))
    hb = h.astype(jnp.bfloat16)
    for k in range(K):
        y_ref[k] = jnp.dot(hb, w_ref[k].astype(jnp.bfloat16),
                           preferred_element_type=jnp.float32)


def _mid(parts, g, b, W):
    return pl.pallas_call(
        _mid_body,
        grid=(NB,),
        in_specs=[
            pl.BlockSpec((NC, BN, CH2), lambda n: (0, n, 0)),
            pl.BlockSpec((1, C), lambda n: (0, 0)),
            pl.BlockSpec((1, C), lambda n: (0, 0)),
            pl.BlockSpec((K, C, C), lambda n: (0, 0, 0)),
        ],
        out_specs=pl.BlockSpec((K, BN, C), lambda n: (0, n, 0)),
        out_shape=jax.ShapeDtypeStruct((K, N, C), jnp.float32),
    )(parts, g, b, W)


def _final_body(p_ref, g_ref, b_ref, x_ref, o_ref):
    h = jnp.concatenate([p_ref[0], p_ref[1]], axis=-1)
    h = _ln(h, g_ref[...], b_ref[...]) + x_ref[...]
    o_ref[...] = jax.nn.gelu(h)


def _final(parts, g, b, x):
    return pl.pallas_call(
        _final_body,
        grid=(NB,),
        in_specs=[
            pl.BlockSpec((NC, BN, CH2), lambda n: (0, n, 0)),
            pl.BlockSpec((1, C), lambda n: (0, 0)),
            pl.BlockSpec((1, C), lambda n: (0, 0)),
            pl.BlockSpec((BN, C), lambda n: (n, 0)),
        ],
        out_specs=pl.BlockSpec((BN, C), lambda n: (n, 0)),
        out_shape=jax.ShapeDtypeStruct((N, C), jnp.float32),
    )(parts, g, b, x)


@functools.cache
def _make_sc_conv():
    mesh = plsc.VectorSubcoreMesh(core_axis_name="c", subcore_axis_name="s")

    @functools.partial(
        pl.kernel,
        out_type=jax.ShapeDtypeStruct((NC, NP, CH2), jnp.float32),
        mesh=mesh,
        compiler_params=pltpu.CompilerParams(use_tc_tiling_on_sc=False),
        scratch_types=[
            pltpu.VMEM((2, CH, G), jnp.int32),
            pltpu.VMEM((G, CH2), jnp.float32),
            pltpu.VMEM((G, CH2), jnp.float32),
            pltpu.VMEM_SHARED((NP, CH2), jnp.float32),
            pltpu.SemaphoreType.DMA,
            pltpu.SemaphoreType.DMA,
        ],
    )
    def sc_conv(y_hbm, eidx_hbm, zeros_hbm, out_hbm,
                idx_v, slot0, slot1, acc_s, sem0, sem1):
        c = lax.axis_index("c")
        s = lax.axis_index("s")
        wid = c * NS + s
        pltpu.sync_copy(zeros_hbm.at[pl.ds(s * RPT, RPT)],
                        acc_s.at[pl.ds(s * RPT, RPT)])
        pltpu.sync_copy(eidx_hbm.at[wid], idx_v)
        plsc.subcore_barrier()

        gix = idx_v.at[0]
        dix = idx_v.at[1]
        pltpu.async_copy(y_hbm.at[gix.at[0]], slot0, sem0)
        pltpu.async_copy(y_hbm.at[gix.at[1]], slot1, sem1)

        def chunk(jj, carry):
            j = 2 * jj
            jp2 = lax.rem(j + 2, CH)
            jp3 = lax.rem(j + 3, CH)
            pltpu.make_async_copy(y_hbm.at[gix.at[0]], slot0, sem0).wait()
            pltpu.sync_copy(slot0, acc_s.at[dix.at[j]], add=True)
            pltpu.async_copy(y_hbm.at[gix.at[jp2]], slot0, sem0)
            pltpu.make_async_copy(y_hbm.at[gix.at[1]], slot1, sem1).wait()
            pltpu.sync_copy(slot1, acc_s.at[dix.at[j + 1]], add=True)
            pltpu.async_copy(y_hbm.at[gix.at[jp3]], slot1, sem1)
            return carry

        lax.fori_loop(0, CH // 2, chunk, 0)
        pltpu.make_async_copy(y_hbm.at[gix.at[0]], slot0, sem0).wait()
        pltpu.make_async_copy(y_hbm.at[gix.at[1]], slot1, sem1).wait()
        plsc.subcore_barrier()
        pltpu.sync_copy(acc_s.at[pl.ds(s * RPT, RPT)],
                        out_hbm.at[c, pl.ds(s * RPT, RPT)])

    return sc_conv


def _sc_conv(y, eidx, zeros):
    return _make_sc_conv()(y, eidx, zeros)


def kernel(x, edge_index, kernel_ids, W1, g1, b1, W2, g2, b2):
    src = edge_index[0].astype(jnp.int32)
    dst = edge_index[1].astype(jnp.int32)
    gidx = 2 * (kernel_ids.astype(jnp.int32) * N + src)
    gidx = gidx.reshape(1, NS, 1, CH, G)
    gidx = jnp.concatenate([gidx, gidx + 1], axis=0)
    didx = jnp.broadcast_to(dst.reshape(1, NS, 1, CH, G), (NC, NS, 1, CH, G))
    eidx = jnp.concatenate([gidx, didx], axis=2).reshape(NW, 2, CH, G)
    zeros = jnp.zeros((NP, CH2), jnp.float32)
    g1r, b1r = g1.reshape(1, C), b1.reshape(1, C)
    g2r, b2r = g2.reshape(1, C), b2.reshape(1, C)

    y1 = _transform(x, W1).reshape(NC * K * N, CH2)
    p1 = _sc_conv(y1, eidx, zeros)
    y2 = _mid(p1, g1r, b1r, W2).reshape(NC * K * N, CH2)
    p2 = _sc_conv(y2, eidx, zeros)
    return _final(p2, g2r, b2r, x)


# speedup vs baseline: 1.9238x; 1.3893x over previous
import functools

import jax
import jax.numpy as jnp
from jax import lax
from jax.experimental import pallas as pl
from jax.experimental.pallas import tpu as pltpu
from jax.experimental.pallas import tpu_sc as plsc

N = 10000
E = 320000
C = 128
K = 9

NC = 2
NS = 16
NW = NC * NS
CH2 = C // NC
EPT = E // NS
G = 125
CH = EPT // G
NP = 10240
RPT = NP // NS

BN = 1000
NB = N // BN


def _mm_body(x_ref, w_ref, y_ref):
    xb = x_ref[...].astype(jnp.bfloat16)
    for k in range(K):
        y_ref[k] = jnp.dot(xb, w_ref[k].astype(jnp.bfloat16),
                           preferred_element_type=jnp.float32)


def _transform(h, W):
    return pl.pallas_call(
        _mm_body,
        grid=(NB,),
        in_specs=[
            pl.BlockSpec((BN, C), lambda n: (n, 0)),
            pl.BlockSpec((K, C, C), lambda n: (0, 0, 0)),
        ],
        out_specs=pl.BlockSpec((K, BN, C), lambda n: (0, n, 0)),
        out_shape=jax.ShapeDtypeStruct((K, N, C), jnp.float32),
    )(h, W)


def _ln(h, g, b):
    mu = jnp.mean(h, axis=-1, keepdims=True)
    var = jnp.mean((h - mu) ** 2, axis=-1, keepdims=True)
    return (h - mu) * lax.rsqrt(var + 1e-6) * g + b


def _mid_body(p_ref, g_ref, b_ref, w_ref, y_ref):
    h = jnp.concatenate([p_ref[0], p_ref[1]], axis=-1)
    h = jax.nn.gelu(_ln(h, g_ref[...], b_ref[...]))
    hb = h.astype(jnp.bfloat16)
    for k in range(K):
        y_ref[k] = jnp.dot(hb, w_ref[k].astype(jnp.bfloat16),
                           preferred_element_type=jnp.float32)


def _mid(parts, g, b, W):
    return pl.pallas_call(
        _mid_body,
        grid=(NB,),
        in_specs=[
            pl.BlockSpec((NC, BN, CH2), lambda n: (0, n, 0)),
            pl.BlockSpec((1, C), lambda n: (0, 0)),
            pl.BlockSpec((1, C), lambda n: (0, 0)),
            pl.BlockSpec((K, C, C), lambda n: (0, 0, 0)),
        ],
        out_specs=pl.BlockSpec((K, BN, C), lambda n: (0, n, 0)),
        out_shape=jax.ShapeDtypeStruct((K, N, C), jnp.float32),
    )(parts, g, b, W)


def _final_body(p_ref, g_ref, b_ref, x_ref, o_ref):
    h = jnp.concatenate([p_ref[0], p_ref[1]], axis=-1)
    h = _ln(h, g_ref[...], b_ref[...]) + x_ref[...]
    o_ref[...] = jax.nn.gelu(h)


def _final(parts, g, b, x):
    return pl.pallas_call(
        _final_body,
        grid=(NB,),
        in_specs=[
            pl.BlockSpec((NC, BN, CH2), lambda n: (0, n, 0)),
            pl.BlockSpec((1, C), lambda n: (0, 0)),
            pl.BlockSpec((1, C), lambda n: (0, 0)),
            pl.BlockSpec((BN, C), lambda n: (n, 0)),
        ],
        out_specs=pl.BlockSpec((BN, C), lambda n: (n, 0)),
        out_shape=jax.ShapeDtypeStruct((N, C), jnp.float32),
    )(parts, g, b, x)


@functools.cache
def _make_sc_conv():
    mesh = plsc.VectorSubcoreMesh(core_axis_name="c", subcore_axis_name="s")

    @functools.partial(
        pl.kernel,
        out_type=jax.ShapeDtypeStruct((NC, NP, CH2), jnp.float32),
        mesh=mesh,
        compiler_params=pltpu.CompilerParams(use_tc_tiling_on_sc=False),
        scratch_types=[
            pltpu.VMEM((2, CH, G), jnp.int32),
            pltpu.VMEM((G, CH2), jnp.float32),
            pltpu.VMEM((G, CH2), jnp.float32),
            pltpu.VMEM((G, CH2), jnp.float32),
            pltpu.VMEM((G, CH2), jnp.float32),
            pltpu.VMEM_SHARED((NP, CH2), jnp.float32),
            pltpu.SemaphoreType.DMA,
            pltpu.SemaphoreType.DMA,
            pltpu.SemaphoreType.DMA,
            pltpu.SemaphoreType.DMA,
        ],
    )
    def sc_conv(y_hbm, eidx_hbm, zeros_hbm, out_hbm,
                idx_v, slot0, slot1, slot2, slot3, acc_s,
                sem0, sem1, sem2, sem3):
        c = lax.axis_index("c")
        s = lax.axis_index("s")
        wid = c * NS + s
        pltpu.sync_copy(zeros_hbm.at[pl.ds(s * RPT, RPT)],
                        acc_s.at[pl.ds(s * RPT, RPT)])
        pltpu.sync_copy(eidx_hbm.at[wid], idx_v)
        plsc.subcore_barrier()

        gix = idx_v.at[0]
        dix = idx_v.at[1]
        H = CH // 2
        pltpu.async_copy(y_hbm.at[gix.at[0]], slot0, sem0)
        pltpu.async_copy(y_hbm.at[gix.at[1]], slot1, sem1)
        pltpu.async_copy(y_hbm.at[gix.at[H]], slot2, sem2)
        pltpu.async_copy(y_hbm.at[gix.at[H + 1]], slot3, sem3)

        def chunk(jj, carry):
            j = 2 * jj
            jp2 = lax.rem(j + 2, H)
            jp3 = lax.rem(j + 3, H)
            pltpu.make_async_copy(y_hbm.at[gix.at[0]], slot0, sem0).wait()
            pltpu.sync_copy(slot0, acc_s.at[dix.at[j]], add=True)
            pltpu.async_copy(y_hbm.at[gix.at[jp2]], slot0, sem0)
            pltpu.make_async_copy(y_hbm.at[gix.at[1]], slot1, sem1).wait()
            pltpu.sync_copy(slot1, acc_s.at[dix.at[j + 1]], add=True)
            pltpu.async_copy(y_hbm.at[gix.at[jp3]], slot1, sem1)
            pltpu.make_async_copy(y_hbm.at[gix.at[0]], slot2, sem2).wait()
            pltpu.sync_copy(slot2, acc_s.at[dix.at[H + j]], add=True)
            pltpu.async_copy(y_hbm.at[gix.at[H + jp2]], slot2, sem2)
            pltpu.make_async_copy(y_hbm.at[gix.at[1]], slot3, sem3).wait()
            pltpu.sync_copy(slot3, acc_s.at[dix.at[H + j + 1]], add=True)
            pltpu.async_copy(y_hbm.at[gix.at[H + jp3]], slot3, sem3)
            return carry

        lax.fori_loop(0, H // 2, chunk, 0)
        pltpu.make_async_copy(y_hbm.at[gix.at[0]], slot0, sem0).wait()
        pltpu.make_async_copy(y_hbm.at[gix.at[1]], slot1, sem1).wait()
        pltpu.make_async_copy(y_hbm.at[gix.at[0]], slot2, sem2).wait()
        pltpu.make_async_copy(y_hbm.at[gix.at[1]], slot3, sem3).wait()
        plsc.subcore_barrier()
        pltpu.sync_copy(acc_s.at[pl.ds(s * RPT, RPT)],
                        out_hbm.at[c, pl.ds(s * RPT, RPT)])

    return sc_conv


def _sc_conv(y, eidx, zeros):
    return _make_sc_conv()(y, eidx, zeros)


def kernel(x, edge_index, kernel_ids, W1, g1, b1, W2, g2, b2):
    src = edge_index[0].astype(jnp.int32)
    dst = edge_index[1].astype(jnp.int32)
    gidx = 2 * (kernel_ids.astype(jnp.int32) * N + src)
    gidx = gidx.reshape(1, NS, 1, CH, G)
    gidx = jnp.concatenate([gidx, gidx + 1], axis=0)
    didx = jnp.broadcast_to(dst.reshape(1, NS, 1, CH, G), (NC, NS, 1, CH, G))
    eidx = jnp.concatenate([gidx, didx], axis=2).reshape(NW, 2, CH, G)
    zeros = jnp.zeros((NP, CH2), jnp.float32)
    g1r, b1r = g1.reshape(1, C), b1.reshape(1, C)
    g2r, b2r = g2.reshape(1, C), b2.reshape(1, C)

    y1 = _transform(x, W1).reshape(NC * K * N, CH2)
    p1 = _sc_conv(y1, eidx, zeros)
    y2 = _mid(p1, g1r, b1r, W2).reshape(NC * K * N, CH2)
    p2 = _sc_conv(y2, eidx, zeros)
    return _final(p2, g2r, b2r, x)
